# Initial kernel scaffold; baseline (speedup 1.0000x reference)
import jax
import jax.numpy as jnp
from jax.experimental import pallas as pl


def kernel(x, edge_index, batch, W1_0, b1_0, W2_0, b2_0, gamma_0, beta_0, W1_1, b1_1, W2_1, b2_1, gamma_1, beta_1, W1_2, b1_2, W2_2, b2_2, gamma_2, beta_2, Wf, bf):
    raise NotImplementedError("write your pallas kernel here")


# speedup vs baseline: 8.5929x; 8.5929x over previous
import functools

import jax
import jax.numpy as jnp
from jax import lax
from jax.experimental import pallas as pl
from jax.experimental.pallas import tpu as pltpu
from jax.experimental.pallas import tpu_sc as plsc

N = 10000
E = 320000
D = 128
H = 64
G = 64
NW = 32
WIN = 128
K = 79
EPAD = NW * K * WIN
NPAD = 10240
BN_EPS = 1e-5

_mesh = plsc.VectorSubcoreMesh(core_axis_name="c", subcore_axis_name="s")


@functools.partial(
    pl.kernel,
    mesh=_mesh,
    out_type=jax.ShapeDtypeStruct((2, NPAD, H), jnp.float32),
    scratch_types=[
        pltpu.VMEM_SHARED((NPAD, H), jnp.float32),
        pltpu.VMEM((K, WIN), jnp.int32),
        pltpu.VMEM((K, WIN), jnp.int32),
        pltpu.VMEM((WIN, H), jnp.float32),
        pltpu.SemaphoreType.DMA,
    ],
    compiler_params=pltpu.CompilerParams(use_tc_tiling_on_sc=False),
)
def _sc_agg(y_hbm, src_hbm, dst_hbm, zeros_hbm, out_hbm,
            acc_sh, src_v, dst_v, rows_v, sem):
    c = lax.axis_index("c")
    s = lax.axis_index("s")
    w = c * 16 + s
    pltpu.sync_copy(zeros_hbm.at[pl.ds(s * 640, 640)],
                    acc_sh.at[pl.ds(s * 640, 640)])
    pltpu.sync_copy(src_hbm.at[w], src_v)
    pltpu.sync_copy(dst_hbm.at[w], dst_v)
    plsc.subcore_barrier()

    @pl.loop(0, K)
    def _(j):
        pltpu.async_copy(y_hbm.at[src_v.at[j]], rows_v, sem).wait()
        pltpu.sync_copy(rows_v, acc_sh.at[dst_v.at[j]], add=True)

    plsc.subcore_barrier()
    pltpu.sync_copy(acc_sh.at[pl.ds(s * 640, 640)],
                    out_hbm.at[c, pl.ds(s * 640, 640)])


def _tc_head_body(x_ref, w1_ref, b1_ref, o_ref):
    o_ref[...] = (jnp.dot(x_ref[...], w1_ref[...],
                          preferred_element_type=jnp.float32,
                          precision=lax.Precision.HIGHEST) + b1_ref[...])


def _layer_tail(y, agg0, agg1, w2, b2, g, be):
    z = jnp.maximum(y + agg0 + agg1, 0.0)
    z = jnp.maximum(
        jnp.dot(z, w2, preferred_element_type=jnp.float32,
                precision=lax.Precision.HIGHEST) + b2, 0.0)
    m1 = jnp.sum(z, axis=0, keepdims=True) * (1.0 / N)
    d = z - m1
    var = jnp.sum(d * d, axis=0, keepdims=True) * (1.0 / N)
    zn = d * lax.rsqrt(var + BN_EPS) * g + be
    return jnp.maximum(zn, 0.0)


def _tc_mid_body(y_ref, agg_ref, w2_ref, b2_ref, g_ref, be_ref,
                 w1n_ref, b1n_ref, o_ref):
    h = _layer_tail(y_ref[...], agg_ref[0, :N], agg_ref[1, :N],
                    w2_ref[...], b2_ref[...], g_ref[...], be_ref[...])
    o_ref[...] = (jnp.dot(h, w1n_ref[...],
                          preferred_element_type=jnp.float32,
                          precision=lax.Precision.HIGHEST) + b1n_ref[...])


def _tc_fin_body(y_ref, agg_ref, w2_ref, b2_ref, g_ref, be_ref,
                 batch_ref, wf_ref, bf_ref, o_ref):
    h = _layer_tail(y_ref[...], agg_ref[0, :N], agg_ref[1, :N],
                    w2_ref[...], b2_ref[...], g_ref[...], be_ref[...])
    iota = lax.broadcasted_iota(jnp.int32, (N, G), 1)
    oh = (iota == batch_ref[...]).astype(jnp.float32)
    pooled = lax.dot_general(oh, h, (((0,), (0,)), ((), ())),
                             preferred_element_type=jnp.float32,
                             precision=lax.Precision.HIGHEST)
    o_ref[...] = (jnp.dot(pooled, wf_ref[...],
                          preferred_element_type=jnp.float32,
                          precision=lax.Precision.HIGHEST) + bf_ref[...])


def _call(body, out_shape, *args):
    return pl.pallas_call(
        body, out_shape=jax.ShapeDtypeStruct(out_shape, jnp.float32))(*args)


def kernel(x, edge_index, batch,
           W1_0, b1_0, W2_0, b2_0, gamma_0, beta_0,
           W1_1, b1_1, W2_1, b2_1, gamma_1, beta_1,
           W1_2, b1_2, W2_2, b2_2, gamma_2, beta_2,
           Wf, bf):
    src = edge_index[0].astype(jnp.int32)
    dst = edge_index[1].astype(jnp.int32)
    pad = EPAD - E
    psrc = (jnp.arange(pad, dtype=jnp.int32) * 97) % N
    pdst = N + (jnp.arange(pad, dtype=jnp.int32) % (NPAD - N))
    srcb = jnp.concatenate([src, psrc]).reshape(NW, K, WIN)
    dstb = jnp.concatenate([dst, pdst]).reshape(NW, K, WIN)
    zeros = jnp.zeros((NPAD, H), jnp.float32)
    batch2 = batch.astype(jnp.int32).reshape(N, 1)

    b1 = [b1_0.reshape(1, H), b1_1.reshape(1, H), b1_2.reshape(1, H)]
    b2 = [b2_0.reshape(1, H), b2_1.reshape(1, H), b2_2.reshape(1, H)]
    gm = [gamma_0.reshape(1, H), gamma_1.reshape(1, H), gamma_2.reshape(1, H)]
    bt = [beta_0.reshape(1, H), beta_1.reshape(1, H), beta_2.reshape(1, H)]
    W2 = [W2_0, W2_1, W2_2]

    y = _call(_tc_head_body, (N, H), x, W1_0, b1[0])
    a = _sc_agg(y, srcb, dstb, zeros)
    y = _call(_tc_mid_body, (N, H), y, a, W2[0], b2[0], gm[0], bt[0],
              W1_1, b1[1])
    a = _sc_agg(y, srcb, dstb, zeros)
    y = _call(_tc_mid_body, (N, H), y, a, W2[1], b2[1], gm[1], bt[1],
              W1_2, b1[2])
    a = _sc_agg(y, srcb, dstb, zeros)
    out = _call(_tc_fin_body, (G, 1), y, a, W2[2], b2[2], gm[2], bt[2],
                batch2, Wf, bf.reshape(1, 1))
    return out
